# Initial kernel scaffold; baseline (speedup 1.0000x reference)
import jax
import jax.numpy as jnp
from jax.experimental import pallas as pl


def kernel(input1, input2):
    raise NotImplementedError("write your pallas kernel here")


# speedup vs baseline: 101.9311x; 101.9311x over previous
import jax
import jax.numpy as jnp
from jax import lax
from jax.experimental import pallas as pl


def _body(z_ref, u_ref, o_ref):
    z = z_ref[...]
    u = u_ref[...]
    mask = u > 0.0
    neg_inf = jnp.float32(-jnp.inf)
    zmax = jnp.max(jnp.where(mask, z, neg_inf), axis=-1, keepdims=True)
    ez = jnp.where(mask, jnp.exp(z - zmax), 0.0)
    Z = jnp.sum(ez, axis=-1, keepdims=True)

    def cond(state):
        tau, prev, k = state
        return jnp.logical_and(k < 64, jnp.any(tau != prev))

    def step(state):
        tau, _, k = state
        A = ez > tau * u
        E = jnp.sum(jnp.where(A, ez, 0.0), axis=-1, keepdims=True)
        U = jnp.sum(jnp.where(A, u, 0.0), axis=-1, keepdims=True)
        new_tau = jnp.maximum(Z - E, 0.0) / jnp.maximum(1.0 - U, 1e-30)
        return new_tau, tau, k + 1

    first = step((Z, Z, 0))
    tau, _, _ = lax.while_loop(cond, step, first)
    tau = jnp.maximum(tau, 1e-30)
    o_ref[...] = jnp.where(mask, jnp.minimum(u, ez / tau), 0.0)


def kernel(input1, input2):
    return pl.pallas_call(
        _body,
        out_shape=jax.ShapeDtypeStruct(input1.shape, input1.dtype),
    )(input1, input2)
